# baseline (speedup 1.0000x reference)
import functools

import jax
import jax.numpy as jnp
from jax import lax
from jax.experimental import pallas as pl
from jax.experimental.pallas import tpu as pltpu
from jax.experimental.pallas import tpu_sc as plsc

NUM_USERS = 1000000
NUM_MOVIES = 100000
EMBED_DIM = 64
BATCH = 16384

NC = 2
NS = 16
NW = NC * NS
B_PER_W = BATCH // NW
N_CHUNK = 4
CHUNK = B_PER_W // N_CHUNK
D2 = 2 * EMBED_DIM


def _sc_gather(uidx2d, midx2d, ut2, mt2):
    mesh = plsc.VectorSubcoreMesh(core_axis_name="c", subcore_axis_name="s")
    out_sds = jax.ShapeDtypeStruct((NW * N_CHUNK, CHUNK, D2), jnp.float32)

    @functools.partial(
        pl.kernel,
        out_type=(out_sds, out_sds),
        mesh=mesh,
        scratch_types=[
            pltpu.VMEM((N_CHUNK, CHUNK), jnp.int32),
            pltpu.VMEM((N_CHUNK, CHUNK), jnp.int32),
            pltpu.VMEM((N_CHUNK, CHUNK, D2), jnp.float32),
            pltpu.VMEM((N_CHUNK - 1, CHUNK, D2), jnp.float32),
            pltpu.SemaphoreType.DMA,
            pltpu.SemaphoreType.DMA,
            pltpu.SemaphoreType.DMA,
        ],
    )
    def k(uid_hbm, mid_hbm, ut_hbm, mt_hbm, u_out, m_out, idxu_v, idxm_v,
          bufu_v, bufm_v, sem_g, sem_wu, sem_wm):
        wid = lax.axis_index("s") * NC + lax.axis_index("c")
        base = wid * N_CHUNK
        pltpu.sync_copy(uid_hbm.at[pl.ds(base, N_CHUNK)], idxu_v)
        pltpu.sync_copy(mid_hbm.at[pl.ds(base, N_CHUNK)], idxm_v)
        gu = [pltpu.async_copy(ut_hbm.at[idxu_v.at[j]], bufu_v.at[j], sem_g)
              for j in range(N_CHUNK)]
        gm = [pltpu.async_copy(mt_hbm.at[idxm_v.at[j]], bufm_v.at[j], sem_g)
              for j in range(N_CHUNK - 1)]
        for c in gu:
            c.wait()
        wu = pltpu.async_copy(bufu_v, u_out.at[pl.ds(base, N_CHUNK)], sem_wu)
        for c in gm:
            c.wait()
        wm0 = pltpu.async_copy(
            bufm_v, m_out.at[pl.ds(base, N_CHUNK - 1)], sem_wm)
        wu.wait()
        glast = pltpu.async_copy(
            mt_hbm.at[idxm_v.at[N_CHUNK - 1]], bufu_v.at[0], sem_g)
        glast.wait()
        wm1 = pltpu.async_copy(
            bufu_v.at[0], m_out.at[base + N_CHUNK - 1], sem_wm)
        wm0.wait()
        wm1.wait()

    return k(uidx2d, midx2d, ut2, mt2)


BR = 2048


def _mlp_body(gu_ref, gm_ref, pu_ref, pm_ref, w1u_ref, w1m_ref, b1_ref,
              w2_ref, b2_ref, o_ref):
    xu = jnp.dot(gu_ref[...], w1u_ref[...], preferred_element_type=jnp.float32)
    xm = jnp.dot(gm_ref[...], w1m_ref[...], preferred_element_type=jnp.float32)
    x = (jnp.where(pu_ref[...] == 0, xu[:, :8], xu[:, 8:])
         + jnp.where(pm_ref[...] == 0, xm[:, :8], xm[:, 8:])
         + b1_ref[...])
    h = jnp.maximum(x, 0.0)
    o = jnp.dot(h, w2_ref[...], preferred_element_type=jnp.float32) + b2_ref[...]
    o_ref[...] = jax.nn.sigmoid(o) * 4.0 + 1.0


def _tc_mlp(gu, gm, pu, pm, w1u, w1m, b1, W2, b2):
    grid = (BATCH // BR,)
    return pl.pallas_call(
        _mlp_body,
        grid=grid,
        in_specs=[
            pl.BlockSpec((BR, D2), lambda i: (i, 0)),
            pl.BlockSpec((BR, D2), lambda i: (i, 0)),
            pl.BlockSpec((BR, 1), lambda i: (i, 0)),
            pl.BlockSpec((BR, 1), lambda i: (i, 0)),
            pl.BlockSpec((D2, 16), lambda i: (0, 0)),
            pl.BlockSpec((D2, 16), lambda i: (0, 0)),
            pl.BlockSpec((1, 8), lambda i: (0, 0)),
            pl.BlockSpec((8, 1), lambda i: (0, 0)),
            pl.BlockSpec((1, 1), lambda i: (0, 0)),
        ],
        out_specs=pl.BlockSpec((BR, 1), lambda i: (i, 0)),
        out_shape=jax.ShapeDtypeStruct((BATCH, 1), jnp.float32),
    )(gu, gm, pu, pm, w1u, w1m, b1, W2, b2)


def kernel(user_ids, movie_ids, user_table, movie_table, W1, b1, W2, b2):
    uid = user_ids.astype(jnp.int32)
    mid = movie_ids.astype(jnp.int32)
    uidx2d = (uid >> 1).reshape(NW * N_CHUNK, CHUNK)
    midx2d = (mid >> 1).reshape(NW * N_CHUNK, CHUNK)
    ut2 = user_table.reshape(NUM_USERS // 2, D2)
    mt2 = movie_table.reshape(NUM_MOVIES // 2, D2)
    gu3, gm3 = _sc_gather(uidx2d, midx2d, ut2, mt2)
    gu = gu3.reshape(BATCH, D2)
    gm = gm3.reshape(BATCH, D2)
    pu = (uid & 1).reshape(BATCH, 1)
    pm = (mid & 1).reshape(BATCH, 1)
    zeros = jnp.zeros((EMBED_DIM, 8), jnp.float32)
    w1u = jnp.concatenate(
        [jnp.concatenate([W1[:EMBED_DIM], zeros], axis=0),
         jnp.concatenate([zeros, W1[:EMBED_DIM]], axis=0)], axis=1)
    w1m = jnp.concatenate(
        [jnp.concatenate([W1[EMBED_DIM:], zeros], axis=0),
         jnp.concatenate([zeros, W1[EMBED_DIM:]], axis=0)], axis=1)
    out = _tc_mlp(gu, gm, pu, pm, w1u, w1m, b1.reshape(1, 8), W2,
                  b2.reshape(1, 1))
    return out.reshape(BATCH)


# speedup vs baseline: 1.6627x; 1.6627x over previous
import functools

import jax
import jax.numpy as jnp
from jax import lax
from jax.experimental import pallas as pl
from jax.experimental.pallas import tpu as pltpu
from jax.experimental.pallas import tpu_sc as plsc

NUM_USERS = 1000000
NUM_MOVIES = 100000
EMBED_DIM = 64
BATCH = 16384

NC = 2
NS = 16
NW = NC * NS
B_PER_W = BATCH // NW
N_CHUNK = 4
CHUNK = B_PER_W // N_CHUNK
D2 = 2 * EMBED_DIM
SEG = 2048


def _transpose_body(x_ref, o_ref):
    xt = x_ref[...].T
    o_ref[...] = jnp.concatenate([xt[:SEG], xt[SEG:]], axis=1)


def _tc_stage(xt, n):
    nb = -(-n // (2 * SEG))
    rows = nb * SEG
    return pl.pallas_call(
        _transpose_body,
        grid=(nb,),
        in_specs=[pl.BlockSpec((EMBED_DIM, 2 * SEG), lambda b: (0, b))],
        out_specs=pl.BlockSpec((SEG, D2), lambda b: (b, 0)),
        out_shape=jax.ShapeDtypeStruct((rows, D2), jnp.float32),
    )(xt)


def _sc_gather(uidx2d, midx2d, zu, zm):
    mesh = plsc.VectorSubcoreMesh(core_axis_name="c", subcore_axis_name="s")
    out_sds = jax.ShapeDtypeStruct((NW * N_CHUNK, CHUNK, D2), jnp.float32)

    @functools.partial(
        pl.kernel,
        out_type=(out_sds, out_sds),
        mesh=mesh,
        scratch_types=[
            pltpu.VMEM((N_CHUNK, CHUNK), jnp.int32),
            pltpu.VMEM((N_CHUNK, CHUNK), jnp.int32),
            pltpu.VMEM((N_CHUNK, CHUNK, D2), jnp.float32),
            pltpu.VMEM((N_CHUNK - 1, CHUNK, D2), jnp.float32),
            pltpu.SemaphoreType.DMA,
            pltpu.SemaphoreType.DMA,
            pltpu.SemaphoreType.DMA,
        ],
    )
    def k(uid_hbm, mid_hbm, zu_hbm, zm_hbm, u_out, m_out, idxu_v, idxm_v,
          bufu_v, bufm_v, sem_g, sem_wu, sem_wm):
        wid = lax.axis_index("s") * NC + lax.axis_index("c")
        base = wid * N_CHUNK
        pltpu.sync_copy(uid_hbm.at[pl.ds(base, N_CHUNK)], idxu_v)
        pltpu.sync_copy(mid_hbm.at[pl.ds(base, N_CHUNK)], idxm_v)
        gu = [pltpu.async_copy(zu_hbm.at[idxu_v.at[j]], bufu_v.at[j], sem_g)
              for j in range(N_CHUNK)]
        gm = [pltpu.async_copy(zm_hbm.at[idxm_v.at[j]], bufm_v.at[j], sem_g)
              for j in range(N_CHUNK - 1)]
        for c in gu:
            c.wait()
        wu = pltpu.async_copy(bufu_v, u_out.at[pl.ds(base, N_CHUNK)], sem_wu)
        for c in gm:
            c.wait()
        wm0 = pltpu.async_copy(
            bufm_v, m_out.at[pl.ds(base, N_CHUNK - 1)], sem_wm)
        wu.wait()
        glast = pltpu.async_copy(
            zm_hbm.at[idxm_v.at[N_CHUNK - 1]], bufu_v.at[0], sem_g)
        glast.wait()
        wm1 = pltpu.async_copy(
            bufu_v.at[0], m_out.at[base + N_CHUNK - 1], sem_wm)
        wm0.wait()
        wm1.wait()

    return k(uidx2d, midx2d, zu, zm)


BR = 2048


def _mlp_body(gu_ref, gm_ref, pu_ref, pm_ref, w1u_ref, w1m_ref, b1_ref,
              w2_ref, b2_ref, o_ref):
    gu = gu_ref[...]
    gm = gm_ref[...]
    usel = jnp.where(pu_ref[...] == 0, gu[:, :EMBED_DIM], gu[:, EMBED_DIM:])
    msel = jnp.where(pm_ref[...] == 0, gm[:, :EMBED_DIM], gm[:, EMBED_DIM:])
    x = (jnp.dot(usel, w1u_ref[...], preferred_element_type=jnp.float32)
         + jnp.dot(msel, w1m_ref[...], preferred_element_type=jnp.float32)
         + b1_ref[...])
    h = jnp.maximum(x, 0.0)
    o = jnp.dot(h, w2_ref[...], preferred_element_type=jnp.float32) + b2_ref[...]
    o_ref[...] = jax.nn.sigmoid(o) * 4.0 + 1.0


def _tc_mlp(gu, gm, pu, pm, w1u, w1m, b1, W2, b2):
    grid = (BATCH // BR,)
    return pl.pallas_call(
        _mlp_body,
        grid=grid,
        in_specs=[
            pl.BlockSpec((BR, D2), lambda i: (i, 0)),
            pl.BlockSpec((BR, D2), lambda i: (i, 0)),
            pl.BlockSpec((BR, 1), lambda i: (i, 0)),
            pl.BlockSpec((BR, 1), lambda i: (i, 0)),
            pl.BlockSpec((EMBED_DIM, 8), lambda i: (0, 0)),
            pl.BlockSpec((EMBED_DIM, 8), lambda i: (0, 0)),
            pl.BlockSpec((1, 8), lambda i: (0, 0)),
            pl.BlockSpec((8, 1), lambda i: (0, 0)),
            pl.BlockSpec((1, 1), lambda i: (0, 0)),
        ],
        out_specs=pl.BlockSpec((BR, 1), lambda i: (i, 0)),
        out_shape=jax.ShapeDtypeStruct((BATCH, 1), jnp.float32),
    )(gu, gm, pu, pm, w1u, w1m, b1, W2, b2)


def _stage_index(ids):
    blk = ids >> 11
    half = blk & 1
    p = ((blk >> 1) << 11) | (ids & (SEG - 1))
    return p, half


def kernel(user_ids, movie_ids, user_table, movie_table, W1, b1, W2, b2):
    uid = user_ids.astype(jnp.int32)
    mid = movie_ids.astype(jnp.int32)
    zu = _tc_stage(user_table.T, NUM_USERS)
    zm = _tc_stage(movie_table.T, NUM_MOVIES)
    pu_idx, pu_half = _stage_index(uid)
    pm_idx, pm_half = _stage_index(mid)
    gu3, gm3 = _sc_gather(pu_idx.reshape(NW * N_CHUNK, CHUNK),
                          pm_idx.reshape(NW * N_CHUNK, CHUNK), zu, zm)
    gu = gu3.reshape(BATCH, D2)
    gm = gm3.reshape(BATCH, D2)
    out = _tc_mlp(gu, gm, pu_half.reshape(BATCH, 1), pm_half.reshape(BATCH, 1),
                  W1[:EMBED_DIM], W1[EMBED_DIM:], b1.reshape(1, 8), W2,
                  b2.reshape(1, 1))
    return out.reshape(BATCH)


# speedup vs baseline: 1.7494x; 1.0521x over previous
import functools

import jax
import jax.numpy as jnp
from jax import lax
from jax.experimental import pallas as pl
from jax.experimental.pallas import tpu as pltpu
from jax.experimental.pallas import tpu_sc as plsc

NUM_USERS = 1000000
NUM_MOVIES = 100000
EMBED_DIM = 64
BATCH = 16384

NC = 2
NS = 16
NW = NC * NS
B_PER_W = BATCH // NW
N_CHUNK = 4
CHUNK = B_PER_W // N_CHUNK
D2 = 2 * EMBED_DIM
SEG = 2048


def _transpose_body(x_ref, e0_ref, e1_ref, o_ref):
    x = x_ref[...]
    dn = (((0,), (0,)), ((), ()))
    o_ref[...] = (
        lax.dot_general(x[:, :SEG], e0_ref[...], dn,
                        preferred_element_type=jnp.float32)
        + lax.dot_general(x[:, SEG:], e1_ref[...], dn,
                          preferred_element_type=jnp.float32))


def _tc_stage(xt, n, e0, e1):
    nb = -(-n // (2 * SEG))
    rows = nb * SEG
    return pl.pallas_call(
        _transpose_body,
        grid=(nb,),
        in_specs=[
            pl.BlockSpec((EMBED_DIM, 2 * SEG), lambda b: (0, b)),
            pl.BlockSpec((EMBED_DIM, D2), lambda b: (0, 0)),
            pl.BlockSpec((EMBED_DIM, D2), lambda b: (0, 0)),
        ],
        out_specs=pl.BlockSpec((SEG, D2), lambda b: (b, 0)),
        out_shape=jax.ShapeDtypeStruct((rows, D2), jnp.float32),
    )(xt, e0, e1)


def _sc_gather(uidx2d, midx2d, zu, zm):
    mesh = plsc.VectorSubcoreMesh(core_axis_name="c", subcore_axis_name="s")
    out_sds = jax.ShapeDtypeStruct((NW * N_CHUNK, CHUNK, D2), jnp.float32)

    @functools.partial(
        pl.kernel,
        out_type=(out_sds, out_sds),
        mesh=mesh,
        scratch_types=[
            pltpu.VMEM((N_CHUNK, CHUNK), jnp.int32),
            pltpu.VMEM((N_CHUNK, CHUNK), jnp.int32),
            pltpu.VMEM((N_CHUNK, CHUNK, D2), jnp.float32),
            pltpu.VMEM((N_CHUNK - 1, CHUNK, D2), jnp.float32),
            pltpu.SemaphoreType.DMA,
            pltpu.SemaphoreType.DMA,
            pltpu.SemaphoreType.DMA,
        ],
    )
    def k(uid_hbm, mid_hbm, zu_hbm, zm_hbm, u_out, m_out, idxu_v, idxm_v,
          bufu_v, bufm_v, sem_g, sem_wu, sem_wm):
        wid = lax.axis_index("s") * NC + lax.axis_index("c")
        base = wid * N_CHUNK
        pltpu.sync_copy(uid_hbm.at[pl.ds(base, N_CHUNK)], idxu_v)
        pltpu.sync_copy(mid_hbm.at[pl.ds(base, N_CHUNK)], idxm_v)
        gu = [pltpu.async_copy(zu_hbm.at[idxu_v.at[j]], bufu_v.at[j], sem_g)
              for j in range(N_CHUNK)]
        gm = [pltpu.async_copy(zm_hbm.at[idxm_v.at[j]], bufm_v.at[j], sem_g)
              for j in range(N_CHUNK - 1)]
        for c in gu:
            c.wait()
        wu = pltpu.async_copy(bufu_v, u_out.at[pl.ds(base, N_CHUNK)], sem_wu)
        for c in gm:
            c.wait()
        wm0 = pltpu.async_copy(
            bufm_v, m_out.at[pl.ds(base, N_CHUNK - 1)], sem_wm)
        wu.wait()
        glast = pltpu.async_copy(
            zm_hbm.at[idxm_v.at[N_CHUNK - 1]], bufu_v.at[0], sem_g)
        glast.wait()
        wm1 = pltpu.async_copy(
            bufu_v.at[0], m_out.at[base + N_CHUNK - 1], sem_wm)
        wm0.wait()
        wm1.wait()

    return k(uidx2d, midx2d, zu, zm)


BR = 2048


def _mlp_body(gu_ref, gm_ref, pu_ref, pm_ref, w1u_ref, w1m_ref, b1_ref,
              w2_ref, b2_ref, o_ref):
    gu = gu_ref[...]
    gm = gm_ref[...]
    usel = jnp.where(pu_ref[...] == 0, gu[:, :EMBED_DIM], gu[:, EMBED_DIM:])
    msel = jnp.where(pm_ref[...] == 0, gm[:, :EMBED_DIM], gm[:, EMBED_DIM:])
    x = (jnp.dot(usel, w1u_ref[...], preferred_element_type=jnp.float32)
         + jnp.dot(msel, w1m_ref[...], preferred_element_type=jnp.float32)
         + b1_ref[...])
    h = jnp.maximum(x, 0.0)
    o = jnp.dot(h, w2_ref[...], preferred_element_type=jnp.float32) + b2_ref[...]
    o_ref[...] = jax.nn.sigmoid(o) * 4.0 + 1.0


def _tc_mlp(gu, gm, pu, pm, w1u, w1m, b1, W2, b2):
    grid = (BATCH // BR,)
    return pl.pallas_call(
        _mlp_body,
        grid=grid,
        in_specs=[
            pl.BlockSpec((BR, D2), lambda i: (i, 0)),
            pl.BlockSpec((BR, D2), lambda i: (i, 0)),
            pl.BlockSpec((BR, 1), lambda i: (i, 0)),
            pl.BlockSpec((BR, 1), lambda i: (i, 0)),
            pl.BlockSpec((EMBED_DIM, 8), lambda i: (0, 0)),
            pl.BlockSpec((EMBED_DIM, 8), lambda i: (0, 0)),
            pl.BlockSpec((1, 8), lambda i: (0, 0)),
            pl.BlockSpec((8, 1), lambda i: (0, 0)),
            pl.BlockSpec((1, 1), lambda i: (0, 0)),
        ],
        out_specs=pl.BlockSpec((BR, 1), lambda i: (i, 0)),
        out_shape=jax.ShapeDtypeStruct((BATCH, 1), jnp.float32),
    )(gu, gm, pu, pm, w1u, w1m, b1, W2, b2)


def _stage_index(ids):
    blk = ids >> 11
    half = blk & 1
    p = ((blk >> 1) << 11) | (ids & (SEG - 1))
    return p, half


def kernel(user_ids, movie_ids, user_table, movie_table, W1, b1, W2, b2):
    uid = user_ids.astype(jnp.int32)
    mid = movie_ids.astype(jnp.int32)
    eye = jnp.eye(EMBED_DIM, dtype=jnp.float32)
    zeros = jnp.zeros((EMBED_DIM, EMBED_DIM), jnp.float32)
    e0 = jnp.concatenate([eye, zeros], axis=1)
    e1 = jnp.concatenate([zeros, eye], axis=1)
    zu = _tc_stage(user_table.T, NUM_USERS, e0, e1)
    zm = _tc_stage(movie_table.T, NUM_MOVIES, e0, e1)
    pu_idx, pu_half = _stage_index(uid)
    pm_idx, pm_half = _stage_index(mid)
    gu3, gm3 = _sc_gather(pu_idx.reshape(NW * N_CHUNK, CHUNK),
                          pm_idx.reshape(NW * N_CHUNK, CHUNK), zu, zm)
    gu = gu3.reshape(BATCH, D2)
    gm = gm3.reshape(BATCH, D2)
    out = _tc_mlp(gu, gm, pu_half.reshape(BATCH, 1), pm_half.reshape(BATCH, 1),
                  W1[:EMBED_DIM], W1[EMBED_DIM:], b1.reshape(1, 8), W2,
                  b2.reshape(1, 1))
    return out.reshape(BATCH)


# speedup vs baseline: 2.1898x; 1.2518x over previous
import functools

import jax
import jax.numpy as jnp
from jax import lax
from jax.experimental import pallas as pl
from jax.experimental.pallas import tpu as pltpu
from jax.experimental.pallas import tpu_sc as plsc

NUM_USERS = 1000000
NUM_MOVIES = 100000
EMBED_DIM = 64
BATCH = 16384

NC = 2
NS = 16
NW = NC * NS
B_PER_W = BATCH // NW
N_CHUNK = 4
CHUNK = B_PER_W // N_CHUNK
D2 = 2 * EMBED_DIM
SEG = 4096


def _transpose_body(x_ref, e0_ref, e1_ref, o_ref):
    x = x_ref[...]
    dn = (((0,), (0,)), ((), ()))
    o_ref[...] = (
        lax.dot_general(x[:, :SEG], e0_ref[...], dn,
                        preferred_element_type=jnp.float32)
        + lax.dot_general(x[:, SEG:], e1_ref[...], dn,
                          preferred_element_type=jnp.float32))


def _tc_stage(xt, n, e0, e1):
    nb = -(-n // (2 * SEG))
    rows = nb * SEG
    return pl.pallas_call(
        _transpose_body,
        grid=(nb,),
        in_specs=[
            pl.BlockSpec((EMBED_DIM, 2 * SEG), lambda b: (0, b)),
            pl.BlockSpec((EMBED_DIM, D2), lambda b: (0, 0)),
            pl.BlockSpec((EMBED_DIM, D2), lambda b: (0, 0)),
        ],
        out_specs=pl.BlockSpec((SEG, D2), lambda b: (b, 0)),
        out_shape=jax.ShapeDtypeStruct((rows, D2), jnp.float32),
        compiler_params=pltpu.CompilerParams(
            dimension_semantics=("parallel",)),
    )(xt, e0, e1)


def _sc_gather(uidx2d, midx2d, zu, zm):
    mesh = plsc.VectorSubcoreMesh(core_axis_name="c", subcore_axis_name="s")
    out_sds = jax.ShapeDtypeStruct((NW * N_CHUNK, CHUNK, D2), jnp.float32)

    @functools.partial(
        pl.kernel,
        out_type=(out_sds, out_sds),
        mesh=mesh,
        scratch_types=[
            pltpu.VMEM((N_CHUNK, CHUNK), jnp.int32),
            pltpu.VMEM((N_CHUNK, CHUNK), jnp.int32),
            pltpu.VMEM((N_CHUNK, CHUNK, D2), jnp.float32),
            pltpu.VMEM((N_CHUNK - 1, CHUNK, D2), jnp.float32),
            pltpu.SemaphoreType.DMA,
            pltpu.SemaphoreType.DMA,
            pltpu.SemaphoreType.DMA,
        ],
    )
    def k(uid_hbm, mid_hbm, zu_hbm, zm_hbm, u_out, m_out, idxu_v, idxm_v,
          bufu_v, bufm_v, sem_g, sem_wu, sem_wm):
        wid = lax.axis_index("s") * NC + lax.axis_index("c")
        base = wid * N_CHUNK
        pltpu.sync_copy(uid_hbm.at[pl.ds(base, N_CHUNK)], idxu_v)
        pltpu.sync_copy(mid_hbm.at[pl.ds(base, N_CHUNK)], idxm_v)
        gu = [pltpu.async_copy(zu_hbm.at[idxu_v.at[j]], bufu_v.at[j], sem_g)
              for j in range(N_CHUNK)]
        gm = [pltpu.async_copy(zm_hbm.at[idxm_v.at[j]], bufm_v.at[j], sem_g)
              for j in range(N_CHUNK - 1)]
        for c in gu:
            c.wait()
        wu = pltpu.async_copy(bufu_v, u_out.at[pl.ds(base, N_CHUNK)], sem_wu)
        for c in gm:
            c.wait()
        wm0 = pltpu.async_copy(
            bufm_v, m_out.at[pl.ds(base, N_CHUNK - 1)], sem_wm)
        wu.wait()
        glast = pltpu.async_copy(
            zm_hbm.at[idxm_v.at[N_CHUNK - 1]], bufu_v.at[0], sem_g)
        glast.wait()
        wm1 = pltpu.async_copy(
            bufu_v.at[0], m_out.at[base + N_CHUNK - 1], sem_wm)
        wm0.wait()
        wm1.wait()

    return k(uidx2d, midx2d, zu, zm)


BR = 2048


def _mlp_body(gu_ref, gm_ref, pu_ref, pm_ref, w1u_ref, w1m_ref, b1_ref,
              w2_ref, b2_ref, o_ref):
    gu = gu_ref[...]
    gm = gm_ref[...]
    usel = jnp.where(pu_ref[...] == 0, gu[:, :EMBED_DIM], gu[:, EMBED_DIM:])
    msel = jnp.where(pm_ref[...] == 0, gm[:, :EMBED_DIM], gm[:, EMBED_DIM:])
    x = (jnp.dot(usel, w1u_ref[...], preferred_element_type=jnp.float32)
         + jnp.dot(msel, w1m_ref[...], preferred_element_type=jnp.float32)
         + b1_ref[...])
    h = jnp.maximum(x, 0.0)
    o = jnp.dot(h, w2_ref[...], preferred_element_type=jnp.float32) + b2_ref[...]
    o_ref[...] = jax.nn.sigmoid(o) * 4.0 + 1.0


def _tc_mlp(gu, gm, pu, pm, w1u, w1m, b1, W2, b2):
    grid = (BATCH // BR,)
    return pl.pallas_call(
        _mlp_body,
        grid=grid,
        in_specs=[
            pl.BlockSpec((BR, D2), lambda i: (i, 0)),
            pl.BlockSpec((BR, D2), lambda i: (i, 0)),
            pl.BlockSpec((BR, 1), lambda i: (i, 0)),
            pl.BlockSpec((BR, 1), lambda i: (i, 0)),
            pl.BlockSpec((EMBED_DIM, 8), lambda i: (0, 0)),
            pl.BlockSpec((EMBED_DIM, 8), lambda i: (0, 0)),
            pl.BlockSpec((1, 8), lambda i: (0, 0)),
            pl.BlockSpec((8, 1), lambda i: (0, 0)),
            pl.BlockSpec((1, 1), lambda i: (0, 0)),
        ],
        out_specs=pl.BlockSpec((BR, 1), lambda i: (i, 0)),
        out_shape=jax.ShapeDtypeStruct((BATCH, 1), jnp.float32),
    )(gu, gm, pu, pm, w1u, w1m, b1, W2, b2)


def _stage_index(ids):
    blk = ids >> 12
    half = blk & 1
    p = ((blk >> 1) << 12) | (ids & (SEG - 1))
    return p, half


def kernel(user_ids, movie_ids, user_table, movie_table, W1, b1, W2, b2):
    uid = user_ids.astype(jnp.int32)
    mid = movie_ids.astype(jnp.int32)
    eye = jnp.eye(EMBED_DIM, dtype=jnp.float32)
    zeros = jnp.zeros((EMBED_DIM, EMBED_DIM), jnp.float32)
    e0 = jnp.concatenate([eye, zeros], axis=1)
    e1 = jnp.concatenate([zeros, eye], axis=1)
    zu = _tc_stage(user_table.T, NUM_USERS, e0, e1)
    zm = _tc_stage(movie_table.T, NUM_MOVIES, e0, e1)
    pu_idx, pu_half = _stage_index(uid)
    pm_idx, pm_half = _stage_index(mid)
    gu3, gm3 = _sc_gather(pu_idx.reshape(NW * N_CHUNK, CHUNK),
                          pm_idx.reshape(NW * N_CHUNK, CHUNK), zu, zm)
    gu = gu3.reshape(BATCH, D2)
    gm = gm3.reshape(BATCH, D2)
    out = _tc_mlp(gu, gm, pu_half.reshape(BATCH, 1), pm_half.reshape(BATCH, 1),
                  W1[:EMBED_DIM], W1[EMBED_DIM:], b1.reshape(1, 8), W2,
                  b2.reshape(1, 1))
    return out.reshape(BATCH)
